# baseline (speedup 1.0000x reference)
import jax
import jax.numpy as jnp
from jax.experimental import pallas as pl

N = 10000
NG = 8
NC_OUT = 2


def _head_body(pooled_ref, wm1_ref, bm1_ref, wm2_ref, bm2_ref, out_ref):
    p = pooled_ref[...]
    h1 = jnp.maximum(jnp.dot(p, wm1_ref[...], preferred_element_type=jnp.float32) + bm1_ref[...], 0.0)
    out_ref[...] = jnp.dot(h1, wm2_ref[...], preferred_element_type=jnp.float32) + bm2_ref[...]


def kernel(x, edge_index, edge_attr, batch, W0, b0, g0, be0, W1, b1, g1, be1, W2, b2, g2, be2, Wm1, bm1, Wm2, bm2):
    w = jnp.abs(edge_attr)
    src, dst = edge_index[0], edge_index[1]
    n = x.shape[0]
    loop = jnp.arange(n)
    s = jnp.concatenate([src, loop])
    d = jnp.concatenate([dst, loop])
    ww = jnp.concatenate([w, jnp.ones((n,), w.dtype)])
    deg = jnp.zeros((n,), x.dtype).at[d].add(ww)
    dinv = jnp.where(deg > 0, jax.lax.rsqrt(jnp.maximum(deg, 1e-12)), 0.0)
    norm = dinv[s] * ww * dinv[d]
    z = x
    for (W, b, g, be) in ((W0, b0, g0, be0), (W1, b1, g1, be1), (W2, b2, g2, be2)):
        h = z @ W
        out = jnp.zeros_like(h).at[d].add(norm[:, None] * h[s])
        z = out + b
        z = jnp.where(z > 0, z, 0.2 * z)
        m = z.mean(axis=0)
        v = z.var(axis=0)
        z = (z - m) * jax.lax.rsqrt(v + 1e-5) * g + be
    sums = jax.ops.segment_sum(z, batch, num_segments=NG)
    cnt = jax.ops.segment_sum(jnp.ones((z.shape[0],), z.dtype), batch, num_segments=NG)
    pooled = sums / jnp.maximum(cnt, 1.0)[:, None]
    return pl.pallas_call(
        _head_body,
        out_shape=jax.ShapeDtypeStruct((NG, NC_OUT), jnp.float32),
    )(pooled, Wm1, bm1, Wm2, bm2)


# speedup vs baseline: 8.5063x; 8.5063x over previous
import functools

import jax
import jax.numpy as jnp
from jax import lax
from jax.experimental import pallas as pl
from jax.experimental.pallas import tpu as pltpu
from jax.experimental.pallas import tpu_sc as plsc

N = 10000
NPAD = 10240
E = 320000
D = 128
NG = 8
NCOUT = 2
NW = 32
EPW = E // NW
K = 80
NCHUNK = EPW // K
RPT = NPAD // 16

_mesh = plsc.VectorSubcoreMesh(core_axis_name="c", subcore_axis_name="s")

_GATHER_DNUMS = lax.GatherDimensionNumbers(
    offset_dims=(), collapsed_slice_dims=(0,), start_index_map=(0,))


def _lane_bcast(vec, l):
    idx = jnp.full((16, 1), l, jnp.int32)
    return lax.gather(vec, idx, _GATHER_DNUMS, (1,),
                      mode=lax.GatherScatterMode.PROMISE_IN_BOUNDS)


@functools.partial(
    pl.kernel,
    out_type=jax.ShapeDtypeStruct((2, NPAD), jnp.float32),
    mesh=_mesh,
    scratch_types=[
        pltpu.VMEM((K,), jnp.int32),
        pltpu.VMEM((K,), jnp.float32),
        pltpu.VMEM((RPT,), jnp.float32),
        pltpu.VMEM_SHARED((NPAD,), jnp.float32),
    ],
)
def _sc_deg(dst_hbm, w_hbm, out_hbm, dst_v, w_v, zb_v, acc_sh):
    c = lax.axis_index("c")
    s = lax.axis_index("s")
    wid = s * 2 + c
    for i in range(RPT // 16):
        zb_v[pl.ds(i * 16, 16)] = jnp.zeros((16,), jnp.float32)
    pltpu.sync_copy(zb_v, acc_sh.at[pl.ds(s * RPT, RPT)])
    plsc.subcore_barrier()

    def chunk(ci, carry):
        off = pl.multiple_of(wid * EPW + ci * K, 8)
        pltpu.sync_copy(dst_hbm.at[pl.ds(off, K)], dst_v)
        pltpu.sync_copy(w_hbm.at[pl.ds(off, K)], w_v)

        def absgrp(g, cc):
            w_v[pl.ds(g * 16, 16)] = jnp.abs(w_v[pl.ds(g * 16, 16)])
            return cc

        lax.fori_loop(0, K // 16, absgrp, 0)
        pltpu.sync_copy(w_v, acc_sh.at[dst_v], add=True)
        return carry

    lax.fori_loop(0, NCHUNK, chunk, 0)
    plsc.subcore_barrier()
    pltpu.sync_copy(acc_sh.at[pl.ds(s * RPT, RPT)],
                    out_hbm.at[c, pl.ds(s * RPT, RPT)])


@functools.partial(
    pl.kernel,
    out_type=jax.ShapeDtypeStruct((2, NPAD, D), jnp.float32),
    mesh=_mesh,
    scratch_types=[
        pltpu.VMEM((K,), jnp.int32),
        pltpu.VMEM((K,), jnp.int32),
        pltpu.VMEM((K,), jnp.float32),
        pltpu.VMEM((K, D), jnp.float32),
        pltpu.VMEM_SHARED((NPAD, D), jnp.float32),
        pltpu.SemaphoreType.DMA,
    ],
)
def _sc_edge(hp_hbm, src_hbm, dst_hbm, w_hbm, out_hbm,
             src_v, dst_v, w_v, rows_v, acc_sh, sem):
    c = lax.axis_index("c")
    s = lax.axis_index("s")
    wid = s * 2 + c
    pltpu.sync_copy(hp_hbm.at[pl.ds(s * RPT, RPT)], acc_sh.at[pl.ds(s * RPT, RPT)])
    plsc.subcore_barrier()

    def chunk(ci, carry):
        off = pl.multiple_of(wid * EPW + ci * K, 8)
        pltpu.sync_copy(src_hbm.at[pl.ds(off, K)], src_v)
        pltpu.sync_copy(dst_hbm.at[pl.ds(off, K)], dst_v)
        pltpu.sync_copy(w_hbm.at[pl.ds(off, K)], w_v)
        pltpu.async_copy(hp_hbm.at[src_v], rows_v, sem).wait()

        def grp(g, cc):
            wvec = jnp.abs(w_v[pl.ds(g * 16, 16)])
            base = g * 16
            for l in range(16):
                sv = _lane_bcast(wvec, l)
                e = base + l
                for j in range(D // 16):
                    rows_v[e, pl.ds(j * 16, 16)] = rows_v[e, pl.ds(j * 16, 16)] * sv
            return cc

        lax.fori_loop(0, K // 16, grp, 0)
        pltpu.sync_copy(rows_v, acc_sh.at[dst_v], add=True)
        return carry

    lax.fori_loop(0, NCHUNK, chunk, 0)
    plsc.subcore_barrier()
    pltpu.sync_copy(acc_sh.at[pl.ds(s * RPT, RPT)],
                    out_hbm.at[c, pl.ds(s * RPT, RPT)])


def _t_dinv_body(dp_ref, out_ref):
    dp = dp_ref[...]
    out_ref[...] = lax.rsqrt(1.0 + dp[0] + dp[1])


def _t_dinv(deg_parts):
    return pl.pallas_call(
        _t_dinv_body,
        out_shape=jax.ShapeDtypeStruct((NPAD,), jnp.float32),
    )(deg_parts)


def _store_h_hp(h, dv, h_ref, hp_ref):
    zpad = jnp.zeros((NPAD - N, D), jnp.float32)
    h_ref[0:N, :] = h
    h_ref[N:NPAD, :] = zpad
    hp_ref[0:N, :] = dv * h
    hp_ref[N:NPAD, :] = zpad


def _t_mm0_body(x_ref, w_ref, dv_ref, h_ref, hp_ref):
    h = jnp.dot(x_ref[...], w_ref[...], preferred_element_type=jnp.float32)
    _store_h_hp(h, dv_ref[0:N, :], h_ref, hp_ref)


_H_OUT = [jax.ShapeDtypeStruct((NPAD, D), jnp.float32),
          jax.ShapeDtypeStruct((NPAD, D), jnp.float32)]


def _t_mm0(x, W, dinv_col):
    return pl.pallas_call(_t_mm0_body, out_shape=_H_OUT)(x, W, dinv_col)


def _bn_block(parts_ref, h_ref, dv_ref, b_ref, g_ref, be_ref):
    p = parts_ref[0, 0:N, :] + parts_ref[1, 0:N, :]
    dv = dv_ref[0:N, :]
    h = h_ref[0:N, :]
    hp = dv * h
    z = dv * (p - hp) + b_ref[...]
    z = jnp.where(z > 0, z, 0.2 * z)
    m = jnp.sum(z, axis=0) * (1.0 / N)
    zc = z - m
    v = jnp.sum(zc * zc, axis=0) * (1.0 / N)
    return zc * lax.rsqrt(v + 1e-5) * g_ref[...] + be_ref[...]


def _t_layer_body(parts_ref, h_ref, dv_ref, b_ref, g_ref, be_ref, w_ref,
                  hn_ref, hpn_ref):
    z = _bn_block(parts_ref, h_ref, dv_ref, b_ref, g_ref, be_ref)
    hn = jnp.dot(z, w_ref[...], preferred_element_type=jnp.float32)
    _store_h_hp(hn, dv_ref[0:N, :], hn_ref, hpn_ref)


def _t_layer(parts, h_prev, dinv_col, b, g, be, W_next):
    return pl.pallas_call(_t_layer_body, out_shape=_H_OUT)(
        parts, h_prev, dinv_col, b, g, be, W_next)


def _t_final_body(parts_ref, h_ref, dv_ref, b_ref, g_ref, be_ref, batch_ref,
                  wm1_ref, bm1_ref, wm2_ref, bm2_ref, out_ref):
    z = _bn_block(parts_ref, h_ref, dv_ref, b_ref, g_ref, be_ref)
    bt = batch_ref[...]
    gid = lax.broadcasted_iota(jnp.int32, (NG, N), 0)
    oh = (gid == bt[None, :]).astype(jnp.float32)
    sums = jnp.dot(oh, z, preferred_element_type=jnp.float32)
    cnt = jnp.dot(oh, jnp.ones((N, 1), jnp.float32),
                  preferred_element_type=jnp.float32)
    pooled = sums / jnp.maximum(cnt, 1.0)
    h1 = jnp.maximum(
        jnp.dot(pooled, wm1_ref[...], preferred_element_type=jnp.float32)
        + bm1_ref[...], 0.0)
    out_ref[...] = (jnp.dot(h1, wm2_ref[...], preferred_element_type=jnp.float32)
                    + bm2_ref[...])


def _t_final(parts, h_prev, dinv_col, b, g, be, batch, Wm1, bm1, Wm2, bm2):
    return pl.pallas_call(
        _t_final_body,
        out_shape=jax.ShapeDtypeStruct((NG, NCOUT), jnp.float32),
    )(parts, h_prev, dinv_col, b, g, be, batch, Wm1, bm1, Wm2, bm2)


def kernel(x, edge_index, edge_attr, batch,
           W0, b0, g0, be0, W1, b1, g1, be1, W2, b2, g2, be2,
           Wm1, bm1, Wm2, bm2):
    src = edge_index[0]
    dst = edge_index[1]
    deg_parts = _sc_deg(dst, edge_attr)
    dinv = _t_dinv(deg_parts)
    dinv_col = dinv[:, None]
    h0, hp0 = _t_mm0(x, W0, dinv_col)
    parts = _sc_edge(hp0, src, dst, edge_attr)
    h1, hp1 = _t_layer(parts, h0, dinv_col, b0, g0, be0, W1)
    parts = _sc_edge(hp1, src, dst, edge_attr)
    h2, hp2 = _t_layer(parts, h1, dinv_col, b1, g1, be1, W2)
    parts = _sc_edge(hp2, src, dst, edge_attr)
    return _t_final(parts, h2, dinv_col, b2, g2, be2, batch, Wm1, bm1, Wm2, bm2)
